# baseline (speedup 1.0000x reference)
import jax
import jax.numpy as jnp
from jax.experimental import pallas as pl


def _fused_kernel(x_ref, Ws1_ref, Wn1_ref, b1_ref, Ws2_ref, Wn2_ref, b2_ref,
                  Wp1_ref, bp1_ref, Wp2_ref, bp2_ref, loc_ref, glob_ref):
    x = x_ref[...]
    B, N, D = x.shape
    inv_deg = 1.0 / (N - 1)
    xf = x.reshape(B * N, D)

    agg1 = ((jnp.sum(x, axis=1)[:, None, :] - x) * inv_deg).reshape(B * N, D)
    t1 = (jnp.dot(xf, Ws1_ref[...], preferred_element_type=jnp.float32)
          + jnp.dot(agg1, Wn1_ref[...], preferred_element_type=jnp.float32)
          + b1_ref[...])
    h = jnp.maximum(t1.reshape(B, N, -1), 0.0)

    hf = h.reshape(B * N, -1)
    agg2 = ((jnp.sum(h, axis=1)[:, None, :] - h) * inv_deg).reshape(B * N, -1)
    t2 = (jnp.dot(hf, Ws2_ref[...], preferred_element_type=jnp.float32)
          + jnp.dot(agg2, Wn2_ref[...], preferred_element_type=jnp.float32)
          + b2_ref[...])
    h2 = jnp.maximum(t2.reshape(B, N, -1), 0.0)

    p = jnp.maximum(jnp.dot(h2.reshape(B * N, -1), Wp1_ref[...],
                            preferred_element_type=jnp.float32)
                    + bp1_ref[...], 0.0)
    out = jnp.dot(p, Wp2_ref[...],
                  preferred_element_type=jnp.float32) + bp2_ref[...]
    out = out.reshape(B, N, -1)

    loc_ref[...] = out
    glob_ref[...] = jnp.mean(out, axis=1)


def kernel(samples, Ws1, Wn1, b1, Ws2, Wn2, b2, Wp1, bp1, Wp2, bp2):
    B, N, _ = samples.shape
    p_o = Wp2.shape[1]
    loc, glob = pl.pallas_call(
        _fused_kernel,
        out_shape=(
            jax.ShapeDtypeStruct((B, N, p_o), jnp.float32),
            jax.ShapeDtypeStruct((B, p_o), jnp.float32),
        ),
    )(samples, Ws1, Wn1, b1.reshape(1, -1), Ws2, Wn2, b2.reshape(1, -1),
      Wp1, bp1.reshape(1, -1), Wp2, bp2.reshape(1, -1))
    return glob[:, None, :], loc


# speedup vs baseline: 1.0623x; 1.0623x over previous
import jax
import jax.numpy as jnp
from jax.experimental import pallas as pl


def _fused_kernel(x_ref, Ws1_ref, Wn1_ref, b1_ref, Ws2_ref, Wn2_ref, b2_ref,
                  Wp1_ref, bp1_ref, Wp2_ref, bp2_ref, loc_ref, glob_ref):
    x = x_ref[...]
    B, N, D = x.shape
    inv_deg = 1.0 / (N - 1)
    xf = x.reshape(B * N, D)

    agg1 = ((jnp.sum(x, axis=1)[:, None, :] - x) * inv_deg).reshape(B * N, D)
    xa1 = jnp.concatenate([xf, agg1], axis=1)
    W1 = jnp.concatenate([Ws1_ref[...], Wn1_ref[...]], axis=0)
    t1 = jnp.dot(xa1, W1, preferred_element_type=jnp.float32) + b1_ref[...]
    h = jnp.maximum(t1.reshape(B, N, -1), 0.0)

    hf = h.reshape(B * N, -1)
    agg2 = ((jnp.sum(h, axis=1)[:, None, :] - h) * inv_deg).reshape(B * N, -1)
    xa2 = jnp.concatenate([hf, agg2], axis=1)
    W2 = jnp.concatenate([Ws2_ref[...], Wn2_ref[...]], axis=0)
    t2 = jnp.dot(xa2, W2, preferred_element_type=jnp.float32) + b2_ref[...]
    h2 = jnp.maximum(t2.reshape(B, N, -1), 0.0)

    p = jnp.maximum(jnp.dot(h2.reshape(B * N, -1), Wp1_ref[...],
                            preferred_element_type=jnp.float32)
                    + bp1_ref[...], 0.0)
    out = jnp.dot(p, Wp2_ref[...],
                  preferred_element_type=jnp.float32) + bp2_ref[...]
    out = out.reshape(B, N, -1)

    loc_ref[...] = out
    glob_ref[...] = jnp.mean(out, axis=1)


def kernel(samples, Ws1, Wn1, b1, Ws2, Wn2, b2, Wp1, bp1, Wp2, bp2):
    B, N, _ = samples.shape
    p_o = Wp2.shape[1]
    loc, glob = pl.pallas_call(
        _fused_kernel,
        out_shape=(
            jax.ShapeDtypeStruct((B, N, p_o), jnp.float32),
            jax.ShapeDtypeStruct((B, p_o), jnp.float32),
        ),
    )(samples, Ws1, Wn1, b1.reshape(1, -1), Ws2, Wn2, b2.reshape(1, -1),
      Wp1, bp1.reshape(1, -1), Wp2, bp2.reshape(1, -1))
    return glob[:, None, :], loc
